# Initial kernel scaffold; baseline (speedup 1.0000x reference)
import jax
import jax.numpy as jnp
from jax.experimental import pallas as pl


def kernel(embedding, p_num, text_len, edge_index, edge_attr, dense_w, dense_b, w1, b1, w2, b2):
    raise NotImplementedError("write your pallas kernel here")


# speedup vs baseline: 9.2163x; 9.2163x over previous
import functools

import jax
import jax.numpy as jnp
from jax import lax
from jax.experimental import pallas as pl
from jax.experimental.pallas import tpu as pltpu
from jax.experimental.pallas import tpu_sc as plsc

N = 10000
E = 320000
H = 768
F1 = 128
F2 = 16
NC = 2
NS = 16
NT = NC * NS
EPT = E // NT
K = 80
NCHUNK = EPT // K
NPAD = 10240
RPT = NPAD // NS
ZR = 128

_mesh = plsc.VectorSubcoreMesh(core_axis_name="c", subcore_axis_name="s",
                               num_cores=NC)


def _deg_body(col_hbm, ew_hbm, out_hbm, col_v, ew_v, zb_v, deg_sp):
    c = lax.axis_index("c")
    s = lax.axis_index("s")
    zsl = NPAD // NS

    def zb(i, carry):
        zb_v[pl.ds(i * 16, 16)] = jnp.zeros((16,), jnp.float32)
        return carry

    lax.fori_loop(0, zsl // 16, zb, 0)
    pltpu.sync_copy(zb_v, deg_sp.at[pl.ds(s * zsl, zsl)])
    plsc.subcore_barrier()

    wid = c * NS + s
    pltpu.sync_copy(col_hbm.at[wid], col_v)
    pltpu.sync_copy(ew_hbm.at[wid], ew_v)

    def chunk(j, carry):
        pltpu.sync_copy(ew_v.at[j], deg_sp.at[col_v.at[j]], add=True)
        return carry

    lax.fori_loop(0, NCHUNK, chunk, 0)
    plsc.subcore_barrier()
    pltpu.sync_copy(deg_sp.at[pl.ds(s * zsl, zsl)],
                    out_hbm.at[c, pl.ds(s * zsl, zsl)])


_deg = pl.kernel(
    _deg_body,
    out_type=jax.ShapeDtypeStruct((NC, NPAD), jnp.float32),
    mesh=_mesh,
    compiler_params=pltpu.CompilerParams(use_tc_tiling_on_sc=False),
    scratch_types=[
        pltpu.VMEM((NCHUNK, K), jnp.int32),
        pltpu.VMEM((NCHUNK, K), jnp.float32),
        pltpu.VMEM((NPAD // NS,), jnp.float32),
        pltpu.VMEM_SHARED((NPAD,), jnp.float32),
    ],
)


FH = F1 // NC
EPT2 = E // NS
NCHUNK2 = EPT2 // K


def _scale_rows(rows_v, ew_v, j, F):
    def scale(g, carry):
        wv = ew_v[j, pl.ds(g * 16, 16)]
        for l in range(16):
            w = wv[l]
            e = g * 16 + l
            for f in range(F // 16):
                sl = pl.ds(f * 16, 16)
                rows_v[e, sl] = rows_v[e, sl] * w
        return carry

    lax.fori_loop(0, K // 16, scale, 0)


def _zero_acc(zb_v, acc_sp, s, F):
    def zb(i, carry):
        for f in range(F // 16):
            zb_v[i, pl.ds(f * 16, 16)] = jnp.zeros((16,), jnp.float32)
        return carry

    lax.fori_loop(0, ZR, zb, 0)
    for t in range(RPT // ZR):
        pltpu.sync_copy(zb_v, acc_sp.at[pl.ds(s * RPT + t * ZR, ZR)])
    plsc.subcore_barrier()


def _agg128_body(z_hbm, row_hbm, col_hbm, ew_hbm, out_hbm,
                 row_v, col_v, ew_v, rows_v, zb_v, acc_sp, sem):
    c = lax.axis_index("c")
    s = lax.axis_index("s")
    _zero_acc(zb_v, acc_sp, s, FH)

    pltpu.sync_copy(row_hbm.at[s], row_v)
    pltpu.sync_copy(col_hbm.at[s], col_v)
    pltpu.sync_copy(ew_hbm.at[s], ew_v)

    def chunk(j, carry):
        pltpu.async_copy(z_hbm.at[c].at[row_v.at[j]], rows_v, sem).wait()
        _scale_rows(rows_v, ew_v, j, FH)
        pltpu.sync_copy(rows_v, acc_sp.at[col_v.at[j]], add=True)
        return carry

    lax.fori_loop(0, NCHUNK2, chunk, 0)
    plsc.subcore_barrier()
    for t in range(RPT // ZR):
        r0 = s * RPT + t * ZR
        pltpu.sync_copy(acc_sp.at[pl.ds(r0, ZR)], out_hbm.at[c, pl.ds(r0, ZR)])


_agg128 = pl.kernel(
    _agg128_body,
    out_type=jax.ShapeDtypeStruct((NC, NPAD, FH), jnp.float32),
    mesh=_mesh,
    compiler_params=pltpu.CompilerParams(use_tc_tiling_on_sc=False),
    scratch_types=[
        pltpu.VMEM((NCHUNK2, K), jnp.int32),
        pltpu.VMEM((NCHUNK2, K), jnp.int32),
        pltpu.VMEM((NCHUNK2, K), jnp.float32),
        pltpu.VMEM((K, FH), jnp.float32),
        pltpu.VMEM((ZR, FH), jnp.float32),
        pltpu.VMEM_SHARED((NPAD, FH), jnp.float32),
        pltpu.SemaphoreType.DMA,
    ],
)


def _agg16_body(z_hbm, row_hbm, col_hbm, ew_hbm, out_hbm,
                row_v, col_v, ew_v, rows_v, zb_v, acc_sp, sem):
    c = lax.axis_index("c")
    s = lax.axis_index("s")
    _zero_acc(zb_v, acc_sp, s, F2)

    wid = c * NS + s
    pltpu.sync_copy(row_hbm.at[wid], row_v)
    pltpu.sync_copy(col_hbm.at[wid], col_v)
    pltpu.sync_copy(ew_hbm.at[wid], ew_v)

    def chunk(j, carry):
        pltpu.async_copy(z_hbm.at[row_v.at[j]], rows_v, sem).wait()
        _scale_rows(rows_v, ew_v, j, F2)
        pltpu.sync_copy(rows_v, acc_sp.at[col_v.at[j]], add=True)
        return carry

    lax.fori_loop(0, NCHUNK, chunk, 0)
    plsc.subcore_barrier()
    for t in range(RPT // ZR):
        r0 = s * RPT + t * ZR
        pltpu.sync_copy(acc_sp.at[pl.ds(r0, ZR)], out_hbm.at[c, pl.ds(r0, ZR)])


_agg16 = pl.kernel(
    _agg16_body,
    out_type=jax.ShapeDtypeStruct((NC, NPAD, F2), jnp.float32),
    mesh=_mesh,
    compiler_params=pltpu.CompilerParams(use_tc_tiling_on_sc=False),
    scratch_types=[
        pltpu.VMEM((NCHUNK, K), jnp.int32),
        pltpu.VMEM((NCHUNK, K), jnp.int32),
        pltpu.VMEM((NCHUNK, K), jnp.float32),
        pltpu.VMEM((K, F2), jnp.float32),
        pltpu.VMEM((ZR, F2), jnp.float32),
        pltpu.VMEM_SHARED((NPAD, F2), jnp.float32),
        pltpu.SemaphoreType.DMA,
    ],
)


MBLK = 1000


def _tc_a_body(emb, dwT, db, w1aT, w1b, w1c, pn, tl, d0, d1, z1_out, dis_out):
    t = jnp.tanh(jnp.dot(emb[...], dwT[...],
                         preferred_element_type=jnp.float32,
                         precision=lax.Precision.HIGHEST) + db[...])
    y = jnp.dot(t, w1aT[...], preferred_element_type=jnp.float32,
                precision=lax.Precision.HIGHEST)
    y = y + pn[...] * w1b[...] + tl[...] * w1c[...]
    deg = d0[...] + d1[...] + 2.0
    dis = jnp.where(deg > 0, lax.rsqrt(deg), 0.0)
    z1_out[...] = dis * y
    dis_out[...] = dis


def _tc_a(emb, dwT, db, w1aT, w1b, w1c, pn, tl, d0, d1):
    return pl.pallas_call(
        _tc_a_body,
        grid=(N // MBLK,),
        in_specs=[
            pl.BlockSpec((MBLK, H), lambda i: (i, 0)),
            pl.BlockSpec((H, H), lambda i: (0, 0)),
            pl.BlockSpec((1, H), lambda i: (0, 0)),
            pl.BlockSpec((H, F1), lambda i: (0, 0)),
            pl.BlockSpec((1, F1), lambda i: (0, 0)),
            pl.BlockSpec((1, F1), lambda i: (0, 0)),
            pl.BlockSpec((MBLK, 1), lambda i: (i, 0)),
            pl.BlockSpec((MBLK, 1), lambda i: (i, 0)),
            pl.BlockSpec((MBLK, 1), lambda i: (i, 0)),
            pl.BlockSpec((MBLK, 1), lambda i: (i, 0)),
        ],
        out_specs=[
            pl.BlockSpec((MBLK, F1), lambda i: (i, 0)),
            pl.BlockSpec((MBLK, 1), lambda i: (i, 0)),
        ],
        out_shape=[
            jax.ShapeDtypeStruct((N, F1), jnp.float32),
            jax.ShapeDtypeStruct((N, 1), jnp.float32),
        ],
    )(emb, dwT, db, w1aT, w1b, w1c, pn, tl, d0, d1)


def _tc_e_body(acc, z1, dis, b1r, w2Tp, z2_out):
    a = jnp.concatenate([acc[0], acc[1]], axis=1)
    h = jnp.maximum(dis[...] * (a + 2.0 * z1[...]) + b1r[...], 0.0)
    y2 = jnp.dot(h, w2Tp[...], preferred_element_type=jnp.float32,
                 precision=lax.Precision.HIGHEST)
    z2_out[...] = dis[...] * y2


def _tc_e(acc, z1, dis, b1r, w2Tp):
    return pl.pallas_call(
        _tc_e_body,
        out_shape=jax.ShapeDtypeStruct((N, F2), jnp.float32),
    )(acc, z1, dis, b1r, w2Tp)


def _tc_g_body(acc, z2, dis, b2p, out):
    a = acc[0] + acc[1]
    out[...] = dis[...] * (a + 2.0 * z2[...]) + b2p[...]


def _tc_g(acc, z2, dis, b2p):
    return pl.pallas_call(
        _tc_g_body,
        out_shape=jax.ShapeDtypeStruct((N, F2), jnp.float32),
    )(acc, z2, dis, b2p)


def kernel(embedding, p_num, text_len, edge_index, edge_attr,
           dense_w, dense_b, w1, b1, w2, b2):
    row3 = edge_index[0].reshape(NT, NCHUNK, K)
    col3 = edge_index[1].reshape(NT, NCHUNK, K)
    ew3 = edge_attr.reshape(NT, NCHUNK, K)
    row2 = edge_index[0].reshape(NS, NCHUNK2, K)
    col2 = edge_index[1].reshape(NS, NCHUNK2, K)
    ew2 = edge_attr.reshape(NS, NCHUNK2, K)

    deg_parts = _deg(col3, ew3)
    d0 = deg_parts[0, :N][:, None]
    d1 = deg_parts[1, :N][:, None]

    dwT = dense_w.T
    w1aT = w1[:, :H].T
    w1b = w1[:, H][None, :]
    w1c = w1[:, H + 1][None, :]
    z1, dis = _tc_a(embedding, dwT, dense_b[None, :], w1aT, w1b, w1c,
                    p_num, text_len, d0, d1)

    z1h = z1.reshape(N, NC, FH).transpose(1, 0, 2)
    acc1 = _agg128(z1h, row2, col2, ew2)[:, :N, :]

    w2Tp = jnp.zeros((F1, F2), jnp.float32).at[:, :w2.shape[0]].set(w2.T)
    z2 = _tc_e(acc1, z1, dis, b1[None, :], w2Tp)

    acc2 = _agg16(z2, row3, col3, ew3)[:, :N, :]

    b2p = jnp.zeros((1, F2), jnp.float32).at[0, :b2.shape[0]].set(b2)
    out16 = _tc_g(acc2, z2, dis, b2p)
    return out16[:, :10]
